# baseline (speedup 1.0000x reference)
import functools

import jax
import jax.numpy as jnp
from jax.experimental import pallas as pl
from jax.experimental.pallas import tpu as pltpu
from jax.experimental.pallas import tpu_sc as plsc

N_INT = 2
NAB = 128
NF = 128
NG = 25
CUTOFF = 5.0
MAXZ = 100
B, A, NN = 8, 512, 64
BA = B * A
LN2 = 0.6931471805599453
TA = 128


_SC_NC, _SC_NS = 2, 16
_NW = _SC_NC * _SC_NS
_EDGES = BA * NN
_PER_W = _EDGES // _NW
_GC = 256


def _gather_body(y_hbm, idx_hbm, out_hbm, idx_v, rows_v,
                 gsem0, gsem1, osem0, osem1):
    wid = jax.lax.axis_index("s") * _SC_NC + jax.lax.axis_index("c")
    base = wid * _PER_W
    n = _PER_W // _GC
    gsems = (gsem0, gsem1)
    osems = (osem0, osem1)
    pltpu.sync_copy(idx_hbm.at[pl.ds(base, _PER_W)], idx_v)

    def gstart(j, b):
        pltpu.async_copy(y_hbm.at[idx_v.at[pl.ds(j * _GC, _GC)]], rows_v.at[b],
                         gsems[b])

    def gwait(j, b):
        pltpu.make_async_copy(y_hbm.at[idx_v.at[pl.ds(j * _GC, _GC)]],
                              rows_v.at[b], gsems[b]).wait()

    def ostart(j, b):
        pltpu.async_copy(rows_v.at[b], out_hbm.at[pl.ds(base + j * _GC, _GC)],
                         osems[b])

    def owait(b):
        pltpu.make_async_copy(rows_v.at[b], out_hbm.at[pl.ds(base, _GC)],
                              osems[b]).wait()

    gstart(0, 0)

    def outer(j2, carry):
        for b in range(2):
            j = j2 * 2 + b
            nb = 1 - b

            @pl.when(j + 1 < n)
            def _():
                @pl.when(j >= 1)
                def _():
                    owait(nb)

                gstart(j + 1, nb)

            gwait(j, b)
            ostart(j, b)
        return carry

    jax.lax.fori_loop(0, n // 2, outer, 0)
    owait(0)
    owait(1)


def _sc_gather(y, idx):
    k = pl.kernel(
        _gather_body,
        out_type=jax.ShapeDtypeStruct((_EDGES, NAB), jnp.float32),
        mesh=plsc.VectorSubcoreMesh(core_axis_name="c", subcore_axis_name="s"),
        compiler_params=pltpu.CompilerParams(needs_layout_passes=False),
        scratch_types=[
            pltpu.VMEM((_PER_W,), jnp.int32),
            pltpu.VMEM((2, _GC, NAB), jnp.float32),
            pltpu.SemaphoreType.DMA,
            pltpu.SemaphoreType.DMA,
            pltpu.SemaphoreType.DMA,
            pltpu.SemaphoreType.DMA,
        ],
    )
    return k(y, idx)


_DC = 256


def _d2_body(px_hbm, py_hbm, pz_hbm, idx_hbm, out_hbm,
             px_v, py_v, pz_v, idx_v, d2_v, sem):
    wid = jax.lax.axis_index("s") * _SC_NC + jax.lax.axis_index("c")
    base = wid * _PER_W
    pltpu.sync_copy(px_hbm, px_v)
    pltpu.sync_copy(py_hbm, py_v)
    pltpu.sync_copy(pz_hbm, pz_v)

    def chunk(j, carry):
        off = base + j * _DC
        pltpu.sync_copy(idx_hbm.at[pl.ds(off, _DC)], idx_v)

        def sub(k, c2):
            idx = idx_v[pl.ds(k * 16, 16)]
            av = jnp.full((16,), (off + k * 16) // NN, dtype=jnp.int32)
            xj = plsc.load_gather(px_v, [idx])
            yj = plsc.load_gather(py_v, [idx])
            zj = plsc.load_gather(pz_v, [idx])
            xi = plsc.load_gather(px_v, [av])
            yi = plsc.load_gather(py_v, [av])
            zi = plsc.load_gather(pz_v, [av])
            dx = xj - xi
            dy = yj - yi
            dz = zj - zi
            d2_v[pl.ds(k * 16, 16)] = dx * dx + dy * dy + dz * dz
            return c2

        jax.lax.fori_loop(0, _DC // 16, sub, 0)
        pltpu.sync_copy(d2_v, out_hbm.at[pl.ds(off, _DC)])
        return carry

    jax.lax.fori_loop(0, _PER_W // _DC, chunk, 0)


def _sc_d2(px, py, pz, idx):
    k = pl.kernel(
        _d2_body,
        out_type=jax.ShapeDtypeStruct((_EDGES,), jnp.float32),
        mesh=plsc.VectorSubcoreMesh(core_axis_name="c", subcore_axis_name="s"),
        compiler_params=pltpu.CompilerParams(needs_layout_passes=False),
        scratch_types=[
            pltpu.VMEM((BA,), jnp.float32),
            pltpu.VMEM((BA,), jnp.float32),
            pltpu.VMEM((BA,), jnp.float32),
            pltpu.VMEM((_DC,), jnp.int32),
            pltpu.VMEM((_DC,), jnp.float32),
            pltpu.SemaphoreType.DMA,
        ],
    )
    return k(px, py, pz, idx)


def _ssp(t):
    return jnp.maximum(t, 0.0) + jnp.log1p(jnp.exp(-jnp.abs(t))) - LN2


def _embed_body(z_ref, emb_ref, win_ref, x_ref, y_ref):
    z = z_ref[...]
    col = jax.lax.broadcasted_iota(jnp.int32, (BA, NAB), 1)
    onehot = (z == col).astype(jnp.float32)
    x = jnp.dot(onehot, emb_ref[...], preferred_element_type=jnp.float32)
    x_ref[...] = x
    y_ref[...] = jnp.dot(x, win_ref[...], preferred_element_type=jnp.float32)


def _embed_call(z_flat, emb_pad, win0):
    return pl.pallas_call(
        _embed_body,
        out_shape=(
            jax.ShapeDtypeStruct((BA, NAB), jnp.float32),
            jax.ShapeDtypeStruct((BA, NAB), jnp.float32),
        ),
    )(z_flat, emb_pad, win0)


def _f2_body(d2_ref, f2_ref):
    width = CUTOFF / (NG - 1)
    coeff = -0.5 / (width * width)
    r = jnp.sqrt(jnp.maximum(d2_ref[...], 1e-10))
    offs = jax.lax.broadcasted_iota(
        jnp.int32, (TA, NN, NG), 2).astype(jnp.float32) * width
    diff = r[:, :, None] - offs
    f = jnp.exp(coeff * (diff * diff))
    f2_ref[...] = f.reshape(TA * NN, NG)


def _f2_call(d2):
    return pl.pallas_call(
        _f2_body,
        grid=(BA // TA,),
        in_specs=[pl.BlockSpec((TA, NN), lambda t: (t, 0))],
        out_specs=pl.BlockSpec((TA * NN, NG), lambda t: (t, 0)),
        out_shape=jax.ShapeDtypeStruct((_EDGES, NG), jnp.float32),
    )(d2)


def _interact_body(f2_ref, yj_ref, x_ref, wfn1_ref, wfn2_ref, wf2out_ref,
                   wdense_ref, winext_ref, xo_ref, yo_ref):
    t1 = _ssp(jnp.dot(f2_ref[...], wfn1_ref[...],
                      preferred_element_type=jnp.float32))
    wf = jnp.dot(t1, wfn2_ref[...], preferred_element_type=jnp.float32)
    prod = wf * yj_ref[...]
    agg = prod.reshape(TA, NN, NAB).sum(axis=1)
    h = _ssp(jnp.dot(agg, wf2out_ref[...], preferred_element_type=jnp.float32))
    v = jnp.dot(h, wdense_ref[...], preferred_element_type=jnp.float32)
    xo = x_ref[...] + v
    xo_ref[...] = xo
    yo_ref[...] = jnp.dot(xo, winext_ref[...], preferred_element_type=jnp.float32)


def _interact_call(f2, yj, x, wfn1, wfn2, wf2out, wdense, winext):
    full = lambda k: pl.BlockSpec((k, NAB), lambda t: (0, 0))
    return pl.pallas_call(
        _interact_body,
        grid=(BA // TA,),
        in_specs=[
            pl.BlockSpec((TA * NN, NG), lambda t: (t, 0)),
            pl.BlockSpec((TA * NN, NAB), lambda t: (t, 0)),
            pl.BlockSpec((TA, NAB), lambda t: (t, 0)),
            full(NG), full(NF), full(NF), full(NAB), full(NAB),
        ],
        out_specs=(
            pl.BlockSpec((TA, NAB), lambda t: (t, 0)),
            pl.BlockSpec((TA, NAB), lambda t: (t, 0)),
        ),
        out_shape=(
            jax.ShapeDtypeStruct((BA, NAB), jnp.float32),
            jax.ShapeDtypeStruct((BA, NAB), jnp.float32),
        ),
    )(f2, yj, x, wfn1, wfn2, wf2out, wdense, winext)


def kernel(atomic_numbers, positions, cell, cell_offset, neighbors, neighbor_mask,
           embedding, Wfn1, bfn1, Wfn2, bfn2, Win2f, Wf2out, bf2out, Wdense, bdense):
    z_flat = atomic_numbers.reshape(BA, 1).astype(jnp.int32)
    emb_pad = jnp.pad(embedding, ((0, NAB - MAXZ), (0, 0)))
    x, y = _embed_call(z_flat, emb_pad, Win2f[0])

    nbr_flat = (neighbors.astype(jnp.int32)
                + (jnp.arange(B, dtype=jnp.int32) * A)[:, None, None]).reshape(BA * NN)
    pos_flat = positions.reshape(BA, 3)
    d2 = _sc_d2(pos_flat[:, 0], pos_flat[:, 1], pos_flat[:, 2],
                nbr_flat).reshape(BA, NN)
    f2 = _f2_call(d2)

    for i in range(N_INT):
        yj = _sc_gather(y, nbr_flat)
        winext = Win2f[i + 1] if i + 1 < N_INT else Win2f[i]
        x, y = _interact_call(f2, yj, x, Wfn1[i], Wfn2[i],
                              Wf2out[i], Wdense[i], winext)
    return x.reshape(B, A, NAB)


# speedup vs baseline: 1.0444x; 1.0444x over previous
import functools

import jax
import jax.numpy as jnp
from jax.experimental import pallas as pl
from jax.experimental.pallas import tpu as pltpu
from jax.experimental.pallas import tpu_sc as plsc

N_INT = 2
NAB = 128
NF = 128
NG = 25
CUTOFF = 5.0
MAXZ = 100
B, A, NN = 8, 512, 64
BA = B * A
LN2 = 0.6931471805599453
TA = 128


_SC_NC, _SC_NS = 2, 16
_NW = _SC_NC * _SC_NS
_EDGES = BA * NN
_PER_W = _EDGES // _NW
_GC = 256


def _gather_body(y_hbm, idx_hbm, out_hbm, idx_v, rows_v,
                 gsem0, gsem1, osem0, osem1):
    wid = jax.lax.axis_index("s") * _SC_NC + jax.lax.axis_index("c")
    base = wid * _PER_W
    n = _PER_W // _GC
    gsems = (gsem0, gsem1)
    osems = (osem0, osem1)
    pltpu.sync_copy(idx_hbm.at[pl.ds(base, _PER_W)], idx_v)

    def gstart(j, b):
        pltpu.async_copy(y_hbm.at[idx_v.at[pl.ds(j * _GC, _GC)]], rows_v.at[b],
                         gsems[b])

    def gwait(j, b):
        pltpu.make_async_copy(y_hbm.at[idx_v.at[pl.ds(j * _GC, _GC)]],
                              rows_v.at[b], gsems[b]).wait()

    def ostart(j, b):
        pltpu.async_copy(rows_v.at[b], out_hbm.at[pl.ds(base + j * _GC, _GC)],
                         osems[b])

    def owait(b):
        pltpu.make_async_copy(rows_v.at[b], out_hbm.at[pl.ds(base, _GC)],
                              osems[b]).wait()

    gstart(0, 0)

    def outer(j2, carry):
        for b in range(2):
            j = j2 * 2 + b
            nb = 1 - b

            @pl.when(j + 1 < n)
            def _():
                @pl.when(j >= 1)
                def _():
                    owait(nb)

                gstart(j + 1, nb)

            gwait(j, b)
            ostart(j, b)
        return carry

    jax.lax.fori_loop(0, n // 2, outer, 0)
    owait(0)
    owait(1)


def _sc_gather(y, idx):
    k = pl.kernel(
        _gather_body,
        out_type=jax.ShapeDtypeStruct((_EDGES, NAB), jnp.float32),
        mesh=plsc.VectorSubcoreMesh(core_axis_name="c", subcore_axis_name="s"),
        compiler_params=pltpu.CompilerParams(needs_layout_passes=False),
        scratch_types=[
            pltpu.VMEM((_PER_W,), jnp.int32),
            pltpu.VMEM((2, _GC, NAB), jnp.float32),
            pltpu.SemaphoreType.DMA,
            pltpu.SemaphoreType.DMA,
            pltpu.SemaphoreType.DMA,
            pltpu.SemaphoreType.DMA,
        ],
    )
    return k(y, idx)


_DC = 256


def _d2_body(px_hbm, py_hbm, pz_hbm, idx_hbm, out_hbm,
             px_v, py_v, pz_v, idx_v, d2_v, sem):
    wid = jax.lax.axis_index("s") * _SC_NC + jax.lax.axis_index("c")
    base = wid * _PER_W
    pltpu.sync_copy(px_hbm, px_v)
    pltpu.sync_copy(py_hbm, py_v)
    pltpu.sync_copy(pz_hbm, pz_v)

    def chunk(j, carry):
        off = base + j * _DC
        pltpu.sync_copy(idx_hbm.at[pl.ds(off, _DC)], idx_v)

        def sub(k, c2):
            idx = idx_v[pl.ds(k * 16, 16)]
            av = jnp.full((16,), (off + k * 16) // NN, dtype=jnp.int32)
            xj = plsc.load_gather(px_v, [idx])
            yj = plsc.load_gather(py_v, [idx])
            zj = plsc.load_gather(pz_v, [idx])
            xi = plsc.load_gather(px_v, [av])
            yi = plsc.load_gather(py_v, [av])
            zi = plsc.load_gather(pz_v, [av])
            dx = xj - xi
            dy = yj - yi
            dz = zj - zi
            d2_v[pl.ds(k * 16, 16)] = dx * dx + dy * dy + dz * dz
            return c2

        jax.lax.fori_loop(0, _DC // 16, sub, 0)
        pltpu.sync_copy(d2_v, out_hbm.at[pl.ds(off, _DC)])
        return carry

    jax.lax.fori_loop(0, _PER_W // _DC, chunk, 0)


def _sc_d2(px, py, pz, idx):
    k = pl.kernel(
        _d2_body,
        out_type=jax.ShapeDtypeStruct((_EDGES,), jnp.float32),
        mesh=plsc.VectorSubcoreMesh(core_axis_name="c", subcore_axis_name="s"),
        compiler_params=pltpu.CompilerParams(needs_layout_passes=False),
        scratch_types=[
            pltpu.VMEM((BA,), jnp.float32),
            pltpu.VMEM((BA,), jnp.float32),
            pltpu.VMEM((BA,), jnp.float32),
            pltpu.VMEM((_DC,), jnp.int32),
            pltpu.VMEM((_DC,), jnp.float32),
            pltpu.SemaphoreType.DMA,
        ],
    )
    return k(px, py, pz, idx)


def _ssp(t):
    return jnp.maximum(t, 0.0) + jnp.log1p(jnp.exp(-jnp.abs(t))) - LN2


def _embed_body(z_ref, emb_ref, win_ref, x_ref, y_ref):
    z = z_ref[...]
    col = jax.lax.broadcasted_iota(jnp.int32, (BA, NAB), 1)
    onehot = (z == col).astype(jnp.float32)
    x = jnp.dot(onehot, emb_ref[...], preferred_element_type=jnp.float32)
    x_ref[...] = x
    y_ref[...] = jnp.dot(x, win_ref[...], preferred_element_type=jnp.float32)


def _embed_call(z_flat, emb_pad, win0):
    return pl.pallas_call(
        _embed_body,
        out_shape=(
            jax.ShapeDtypeStruct((BA, NAB), jnp.float32),
            jax.ShapeDtypeStruct((BA, NAB), jnp.float32),
        ),
    )(z_flat, emb_pad, win0)


def _interact_body(d2_ref, yj_ref, x_ref, wfn1_ref, wfn2_ref, wf2out_ref,
                   wdense_ref, winext_ref, xo_ref, yo_ref):
    width = CUTOFF / (NG - 1)
    coeff = -0.5 / (width * width)
    r = jnp.sqrt(jnp.maximum(d2_ref[...], 1e-10))
    offs = jax.lax.broadcasted_iota(
        jnp.int32, (TA, NN, NG), 2).astype(jnp.float32) * width
    diff = r[:, :, None] - offs
    f = jnp.exp(coeff * (diff * diff))
    f2 = f.reshape(TA * NN, NG)
    t1 = _ssp(jnp.dot(f2, wfn1_ref[...], preferred_element_type=jnp.float32))
    wf = jnp.dot(t1, wfn2_ref[...], preferred_element_type=jnp.float32)
    prod = wf * yj_ref[...]
    agg = prod.reshape(TA, NN, NAB).sum(axis=1)
    h = _ssp(jnp.dot(agg, wf2out_ref[...], preferred_element_type=jnp.float32))
    v = jnp.dot(h, wdense_ref[...], preferred_element_type=jnp.float32)
    xo = x_ref[...] + v
    xo_ref[...] = xo
    yo_ref[...] = jnp.dot(xo, winext_ref[...], preferred_element_type=jnp.float32)


def _interact_call(d2, yj, x, wfn1, wfn2, wf2out, wdense, winext):
    full = lambda k: pl.BlockSpec((k, NAB), lambda t: (0, 0))
    return pl.pallas_call(
        _interact_body,
        grid=(BA // TA,),
        in_specs=[
            pl.BlockSpec((TA, NN), lambda t: (t, 0)),
            pl.BlockSpec((TA * NN, NAB), lambda t: (t, 0)),
            pl.BlockSpec((TA, NAB), lambda t: (t, 0)),
            full(NG), full(NF), full(NF), full(NAB), full(NAB),
        ],
        out_specs=(
            pl.BlockSpec((TA, NAB), lambda t: (t, 0)),
            pl.BlockSpec((TA, NAB), lambda t: (t, 0)),
        ),
        out_shape=(
            jax.ShapeDtypeStruct((BA, NAB), jnp.float32),
            jax.ShapeDtypeStruct((BA, NAB), jnp.float32),
        ),
    )(d2, yj, x, wfn1, wfn2, wf2out, wdense, winext)


def kernel(atomic_numbers, positions, cell, cell_offset, neighbors, neighbor_mask,
           embedding, Wfn1, bfn1, Wfn2, bfn2, Win2f, Wf2out, bf2out, Wdense, bdense):
    z_flat = atomic_numbers.reshape(BA, 1).astype(jnp.int32)
    emb_pad = jnp.pad(embedding, ((0, NAB - MAXZ), (0, 0)))
    x, y = _embed_call(z_flat, emb_pad, Win2f[0])

    nbr_flat = (neighbors.astype(jnp.int32)
                + (jnp.arange(B, dtype=jnp.int32) * A)[:, None, None]).reshape(BA * NN)
    pos_flat = positions.reshape(BA, 3)
    d2 = _sc_d2(pos_flat[:, 0], pos_flat[:, 1], pos_flat[:, 2],
                nbr_flat).reshape(BA, NN)

    for i in range(N_INT):
        yj = _sc_gather(y, nbr_flat)
        winext = Win2f[i + 1] if i + 1 < N_INT else Win2f[i]
        x, y = _interact_call(d2, yj, x, Wfn1[i], Wfn2[i],
                              Wf2out[i], Wdense[i], winext)
    return x.reshape(B, A, NAB)


# speedup vs baseline: 1.0842x; 1.0381x over previous
import functools

import jax
import jax.numpy as jnp
from jax.experimental import pallas as pl
from jax.experimental.pallas import tpu as pltpu
from jax.experimental.pallas import tpu_sc as plsc

N_INT = 2
NAB = 128
NF = 128
NG = 25
CUTOFF = 5.0
MAXZ = 100
B, A, NN = 8, 512, 64
BA = B * A
LN2 = 0.6931471805599453
TA = 128


_SC_NC, _SC_NS = 2, 16
_NW = _SC_NC * _SC_NS
_EDGES = BA * NN
_PER_W = _EDGES // _NW
_GC = 256


def _gather_body(y_hbm, idx_hbm, out_hbm, idx_v, rows_v,
                 gsem0, gsem1, osem0, osem1):
    wid = jax.lax.axis_index("s") * _SC_NC + jax.lax.axis_index("c")
    base = wid * _PER_W
    n = _PER_W // _GC
    gsems = (gsem0, gsem1)
    osems = (osem0, osem1)
    pltpu.sync_copy(idx_hbm.at[pl.ds(base, _PER_W)], idx_v)

    def gstart(j, b):
        pltpu.async_copy(y_hbm.at[idx_v.at[pl.ds(j * _GC, _GC)]], rows_v.at[b],
                         gsems[b])

    def gwait(j, b):
        pltpu.make_async_copy(y_hbm.at[idx_v.at[pl.ds(j * _GC, _GC)]],
                              rows_v.at[b], gsems[b]).wait()

    def ostart(j, b):
        pltpu.async_copy(rows_v.at[b], out_hbm.at[pl.ds(base + j * _GC, _GC)],
                         osems[b])

    def owait(b):
        pltpu.make_async_copy(rows_v.at[b], out_hbm.at[pl.ds(base, _GC)],
                              osems[b]).wait()

    gstart(0, 0)

    def outer(j2, carry):
        for b in range(2):
            j = j2 * 2 + b
            nb = 1 - b

            @pl.when(j + 1 < n)
            def _():
                @pl.when(j >= 1)
                def _():
                    owait(nb)

                gstart(j + 1, nb)

            gwait(j, b)
            ostart(j, b)
        return carry

    jax.lax.fori_loop(0, n // 2, outer, 0)
    owait(0)
    owait(1)


def _sc_gather(y, idx):
    k = pl.kernel(
        _gather_body,
        out_type=jax.ShapeDtypeStruct((_EDGES, NAB), jnp.float32),
        mesh=plsc.VectorSubcoreMesh(core_axis_name="c", subcore_axis_name="s"),
        compiler_params=pltpu.CompilerParams(needs_layout_passes=False),
        scratch_types=[
            pltpu.VMEM((_PER_W,), jnp.int32),
            pltpu.VMEM((2, _GC, NAB), jnp.float32),
            pltpu.SemaphoreType.DMA,
            pltpu.SemaphoreType.DMA,
            pltpu.SemaphoreType.DMA,
            pltpu.SemaphoreType.DMA,
        ],
    )
    return k(y, idx)


_DC = 256


def _d2_body(px_hbm, py_hbm, pz_hbm, idx_hbm, out_hbm,
             px_v, py_v, pz_v, idx_v, d2_v, sem):
    wid = jax.lax.axis_index("s") * _SC_NC + jax.lax.axis_index("c")
    base = wid * _PER_W
    pltpu.sync_copy(px_hbm, px_v)
    pltpu.sync_copy(py_hbm, py_v)
    pltpu.sync_copy(pz_hbm, pz_v)

    def chunk(j, carry):
        off = base + j * _DC
        pltpu.sync_copy(idx_hbm.at[pl.ds(off, _DC)], idx_v)

        def sub(k, c2):
            idx = idx_v[pl.ds(k * 16, 16)]
            av = jnp.full((16,), (off + k * 16) // NN, dtype=jnp.int32)
            xj = plsc.load_gather(px_v, [idx])
            yj = plsc.load_gather(py_v, [idx])
            zj = plsc.load_gather(pz_v, [idx])
            xi = plsc.load_gather(px_v, [av])
            yi = plsc.load_gather(py_v, [av])
            zi = plsc.load_gather(pz_v, [av])
            dx = xj - xi
            dy = yj - yi
            dz = zj - zi
            d2_v[pl.ds(k * 16, 16)] = dx * dx + dy * dy + dz * dz
            return c2

        jax.lax.fori_loop(0, _DC // 16, sub, 0)
        pltpu.sync_copy(d2_v, out_hbm.at[pl.ds(off, _DC)])
        return carry

    jax.lax.fori_loop(0, _PER_W // _DC, chunk, 0)


def _sc_d2(px, py, pz, idx):
    k = pl.kernel(
        _d2_body,
        out_type=jax.ShapeDtypeStruct((_EDGES,), jnp.float32),
        mesh=plsc.VectorSubcoreMesh(core_axis_name="c", subcore_axis_name="s"),
        compiler_params=pltpu.CompilerParams(needs_layout_passes=False),
        scratch_types=[
            pltpu.VMEM((BA,), jnp.float32),
            pltpu.VMEM((BA,), jnp.float32),
            pltpu.VMEM((BA,), jnp.float32),
            pltpu.VMEM((_DC,), jnp.int32),
            pltpu.VMEM((_DC,), jnp.float32),
            pltpu.SemaphoreType.DMA,
        ],
    )
    return k(px, py, pz, idx)


def _gd2_body(y_hbm, idx_hbm, px_hbm, py_hbm, pz_hbm, yj_hbm, d2_hbm,
              idx_v, rows_v, px_v, py_v, pz_v, d2_v,
              gsem0, gsem1, osem0, osem1):
    wid = jax.lax.axis_index("s") * _SC_NC + jax.lax.axis_index("c")
    base = wid * _PER_W
    n = _PER_W // _GC
    gsems = (gsem0, gsem1)
    osems = (osem0, osem1)
    pltpu.sync_copy(idx_hbm.at[pl.ds(base, _PER_W)], idx_v)
    pltpu.sync_copy(px_hbm, px_v)
    pltpu.sync_copy(py_hbm, py_v)
    pltpu.sync_copy(pz_hbm, pz_v)

    def gstart(j, b):
        pltpu.async_copy(y_hbm.at[idx_v.at[pl.ds(j * _GC, _GC)]], rows_v.at[b],
                         gsems[b])

    def gwait(j, b):
        pltpu.make_async_copy(y_hbm.at[idx_v.at[pl.ds(j * _GC, _GC)]],
                              rows_v.at[b], gsems[b]).wait()

    def ostart(j, b):
        pltpu.async_copy(rows_v.at[b], yj_hbm.at[pl.ds(base + j * _GC, _GC)],
                         osems[b])

    def owait(b):
        pltpu.make_async_copy(rows_v.at[b], yj_hbm.at[pl.ds(base, _GC)],
                              osems[b]).wait()

    gstart(0, 0)

    def outer(j2, carry):
        for b in range(2):
            j = j2 * 2 + b
            nb = 1 - b

            @pl.when(j + 1 < n)
            def _():
                @pl.when(j >= 1)
                def _():
                    owait(nb)

                gstart(j + 1, nb)

            def sub(k, c2):
                idx = idx_v[pl.ds(j * _GC + k * 16, 16)]
                av = jnp.full((16,), 0, dtype=jnp.int32) + (
                    (base + j * _GC + k * 16) // NN)
                xj = plsc.load_gather(px_v, [idx])
                yj = plsc.load_gather(py_v, [idx])
                zj = plsc.load_gather(pz_v, [idx])
                xi = plsc.load_gather(px_v, [av])
                yi = plsc.load_gather(py_v, [av])
                zi = plsc.load_gather(pz_v, [av])
                dx = xj - xi
                dy = yj - yi
                dz = zj - zi
                d2_v[pl.ds(k * 16, 16)] = dx * dx + dy * dy + dz * dz
                return c2

            jax.lax.fori_loop(0, _GC // 16, sub, 0)
            pltpu.sync_copy(d2_v, d2_hbm.at[pl.ds(base + j * _GC, _GC)])

            gwait(j, b)
            ostart(j, b)
        return carry

    jax.lax.fori_loop(0, n // 2, outer, 0)
    owait(0)
    owait(1)


def _sc_gather_d2(y, idx, px, py, pz):
    k = pl.kernel(
        _gd2_body,
        out_type=(
            jax.ShapeDtypeStruct((_EDGES, NAB), jnp.float32),
            jax.ShapeDtypeStruct((_EDGES,), jnp.float32),
        ),
        mesh=plsc.VectorSubcoreMesh(core_axis_name="c", subcore_axis_name="s"),
        compiler_params=pltpu.CompilerParams(needs_layout_passes=False),
        scratch_types=[
            pltpu.VMEM((_PER_W,), jnp.int32),
            pltpu.VMEM((2, _GC, NAB), jnp.float32),
            pltpu.VMEM((BA,), jnp.float32),
            pltpu.VMEM((BA,), jnp.float32),
            pltpu.VMEM((BA,), jnp.float32),
            pltpu.VMEM((_GC,), jnp.float32),
            pltpu.SemaphoreType.DMA,
            pltpu.SemaphoreType.DMA,
            pltpu.SemaphoreType.DMA,
            pltpu.SemaphoreType.DMA,
        ],
    )
    return k(y, idx, px, py, pz)


def _ssp(t):
    return jnp.maximum(t, 0.0) + jnp.log1p(jnp.exp(-jnp.abs(t))) - LN2


def _embed_body(z_ref, emb_ref, win_ref, x_ref, y_ref):
    z = z_ref[...]
    col = jax.lax.broadcasted_iota(jnp.int32, (BA, NAB), 1)
    onehot = (z == col).astype(jnp.float32)
    x = jnp.dot(onehot, emb_ref[...], preferred_element_type=jnp.float32)
    x_ref[...] = x
    y_ref[...] = jnp.dot(x, win_ref[...], preferred_element_type=jnp.float32)


def _embed_call(z_flat, emb_pad, win0):
    return pl.pallas_call(
        _embed_body,
        out_shape=(
            jax.ShapeDtypeStruct((BA, NAB), jnp.float32),
            jax.ShapeDtypeStruct((BA, NAB), jnp.float32),
        ),
    )(z_flat, emb_pad, win0)


def _interact_body(d2_ref, yj_ref, x_ref, wfn1_ref, wfn2_ref, wf2out_ref,
                   wdense_ref, winext_ref, xo_ref, yo_ref):
    width = CUTOFF / (NG - 1)
    coeff = -0.5 / (width * width)
    r = jnp.sqrt(jnp.maximum(d2_ref[...], 1e-10))
    offs = jax.lax.broadcasted_iota(
        jnp.int32, (TA, NN, NG), 2).astype(jnp.float32) * width
    diff = r[:, :, None] - offs
    f = jnp.exp(coeff * (diff * diff))
    f2 = f.reshape(TA * NN, NG)
    t1 = _ssp(jnp.dot(f2, wfn1_ref[...], preferred_element_type=jnp.float32))
    wf = jnp.dot(t1, wfn2_ref[...], preferred_element_type=jnp.float32)
    prod = wf * yj_ref[...]
    agg = prod.reshape(TA, NN, NAB).sum(axis=1)
    h = _ssp(jnp.dot(agg, wf2out_ref[...], preferred_element_type=jnp.float32))
    v = jnp.dot(h, wdense_ref[...], preferred_element_type=jnp.float32)
    xo = x_ref[...] + v
    xo_ref[...] = xo
    yo_ref[...] = jnp.dot(xo, winext_ref[...], preferred_element_type=jnp.float32)


def _interact_call(d2, yj, x, wfn1, wfn2, wf2out, wdense, winext):
    full = lambda k: pl.BlockSpec((k, NAB), lambda t: (0, 0))
    return pl.pallas_call(
        _interact_body,
        grid=(BA // TA,),
        in_specs=[
            pl.BlockSpec((TA, NN), lambda t: (t, 0)),
            pl.BlockSpec((TA * NN, NAB), lambda t: (t, 0)),
            pl.BlockSpec((TA, NAB), lambda t: (t, 0)),
            full(NG), full(NF), full(NF), full(NAB), full(NAB),
        ],
        out_specs=(
            pl.BlockSpec((TA, NAB), lambda t: (t, 0)),
            pl.BlockSpec((TA, NAB), lambda t: (t, 0)),
        ),
        out_shape=(
            jax.ShapeDtypeStruct((BA, NAB), jnp.float32),
            jax.ShapeDtypeStruct((BA, NAB), jnp.float32),
        ),
    )(d2, yj, x, wfn1, wfn2, wf2out, wdense, winext)


def kernel(atomic_numbers, positions, cell, cell_offset, neighbors, neighbor_mask,
           embedding, Wfn1, bfn1, Wfn2, bfn2, Win2f, Wf2out, bf2out, Wdense, bdense):
    z_flat = atomic_numbers.reshape(BA, 1).astype(jnp.int32)
    emb_pad = jnp.pad(embedding, ((0, NAB - MAXZ), (0, 0)))
    x, y = _embed_call(z_flat, emb_pad, Win2f[0])

    nbr_flat = (neighbors.astype(jnp.int32)
                + (jnp.arange(B, dtype=jnp.int32) * A)[:, None, None]).reshape(BA * NN)
    pos_flat = positions.reshape(BA, 3)

    for i in range(N_INT):
        if i == 0:
            yj, d2 = _sc_gather_d2(y, nbr_flat, pos_flat[:, 0],
                                   pos_flat[:, 1], pos_flat[:, 2])
            d2 = d2.reshape(BA, NN)
        else:
            yj = _sc_gather(y, nbr_flat)
        winext = Win2f[i + 1] if i + 1 < N_INT else Win2f[i]
        x, y = _interact_call(d2, yj, x, Wfn1[i], Wfn2[i],
                              Wf2out[i], Wdense[i], winext)
    return x.reshape(B, A, NAB)


# speedup vs baseline: 1.0847x; 1.0005x over previous
import functools

import jax
import jax.numpy as jnp
from jax.experimental import pallas as pl
from jax.experimental.pallas import tpu as pltpu
from jax.experimental.pallas import tpu_sc as plsc

N_INT = 2
NAB = 128
NF = 128
NG = 25
CUTOFF = 5.0
MAXZ = 100
B, A, NN = 8, 512, 64
BA = B * A
LN2 = 0.6931471805599453
TA = 128


_SC_NC, _SC_NS = 2, 16
_NW = _SC_NC * _SC_NS
_EDGES = BA * NN
_PER_W = _EDGES // _NW
_GC = 256


def _gather_body(y_hbm, idx_hbm, out_hbm, idx_v, rows_v,
                 gsem0, gsem1, osem0, osem1):
    wid = jax.lax.axis_index("s") * _SC_NC + jax.lax.axis_index("c")
    base = wid * _PER_W
    n = _PER_W // _GC
    gsems = (gsem0, gsem1)
    osems = (osem0, osem1)
    pltpu.sync_copy(idx_hbm.at[pl.ds(base, _PER_W)], idx_v)

    def gstart(j, b):
        pltpu.async_copy(y_hbm.at[idx_v.at[pl.ds(j * _GC, _GC)]], rows_v.at[b],
                         gsems[b])

    def gwait(j, b):
        pltpu.make_async_copy(y_hbm.at[idx_v.at[pl.ds(j * _GC, _GC)]],
                              rows_v.at[b], gsems[b]).wait()

    def ostart(j, b):
        pltpu.async_copy(rows_v.at[b], out_hbm.at[pl.ds(base + j * _GC, _GC)],
                         osems[b])

    def owait(b):
        pltpu.make_async_copy(rows_v.at[b], out_hbm.at[pl.ds(base, _GC)],
                              osems[b]).wait()

    gstart(0, 0)

    def outer(j2, carry):
        for b in range(2):
            j = j2 * 2 + b
            nb = 1 - b

            @pl.when(j + 1 < n)
            def _():
                @pl.when(j >= 1)
                def _():
                    owait(nb)

                gstart(j + 1, nb)

            gwait(j, b)
            ostart(j, b)
        return carry

    jax.lax.fori_loop(0, n // 2, outer, 0)
    owait(0)
    owait(1)


def _sc_gather(y, idx):
    k = pl.kernel(
        _gather_body,
        out_type=jax.ShapeDtypeStruct((_EDGES, NAB), jnp.float32),
        mesh=plsc.VectorSubcoreMesh(core_axis_name="c", subcore_axis_name="s"),
        compiler_params=pltpu.CompilerParams(needs_layout_passes=False),
        scratch_types=[
            pltpu.VMEM((_PER_W,), jnp.int32),
            pltpu.VMEM((2, _GC, NAB), jnp.float32),
            pltpu.SemaphoreType.DMA,
            pltpu.SemaphoreType.DMA,
            pltpu.SemaphoreType.DMA,
            pltpu.SemaphoreType.DMA,
        ],
    )
    return k(y, idx)


def _gd2_body(y_hbm, idx_hbm, px_hbm, py_hbm, pz_hbm, yj_hbm, d2_hbm,
              idx_v, rows_v, px_v, py_v, pz_v, d2_v,
              gsem0, gsem1, osem0, osem1):
    wid = jax.lax.axis_index("s") * _SC_NC + jax.lax.axis_index("c")
    base = wid * _PER_W
    n = _PER_W // _GC
    gsems = (gsem0, gsem1)
    osems = (osem0, osem1)
    pltpu.sync_copy(idx_hbm.at[pl.ds(base, _PER_W)], idx_v)
    pltpu.sync_copy(px_hbm, px_v)
    pltpu.sync_copy(py_hbm, py_v)
    pltpu.sync_copy(pz_hbm, pz_v)

    def gstart(j, b):
        pltpu.async_copy(y_hbm.at[idx_v.at[pl.ds(j * _GC, _GC)]], rows_v.at[b],
                         gsems[b])

    def gwait(j, b):
        pltpu.make_async_copy(y_hbm.at[idx_v.at[pl.ds(j * _GC, _GC)]],
                              rows_v.at[b], gsems[b]).wait()

    def ostart(j, b):
        pltpu.async_copy(rows_v.at[b], yj_hbm.at[pl.ds(base + j * _GC, _GC)],
                         osems[b])

    def owait(b):
        pltpu.make_async_copy(rows_v.at[b], yj_hbm.at[pl.ds(base, _GC)],
                              osems[b]).wait()

    gstart(0, 0)

    def outer(j2, carry):
        for b in range(2):
            j = j2 * 2 + b
            nb = 1 - b

            @pl.when(j + 1 < n)
            def _():
                @pl.when(j >= 1)
                def _():
                    owait(nb)

                gstart(j + 1, nb)

            def sub(k, c2):
                idx = idx_v[pl.ds(j * _GC + k * 16, 16)]
                av = jnp.full((16,), 0, dtype=jnp.int32) + (
                    (base + j * _GC + k * 16) // NN)
                xj = plsc.load_gather(px_v, [idx])
                yj = plsc.load_gather(py_v, [idx])
                zj = plsc.load_gather(pz_v, [idx])
                xi = plsc.load_gather(px_v, [av])
                yi = plsc.load_gather(py_v, [av])
                zi = plsc.load_gather(pz_v, [av])
                dx = xj - xi
                dy = yj - yi
                dz = zj - zi
                d2_v[pl.ds(k * 16, 16)] = dx * dx + dy * dy + dz * dz
                return c2

            jax.lax.fori_loop(0, _GC // 16, sub, 0)
            pltpu.sync_copy(d2_v, d2_hbm.at[pl.ds(base + j * _GC, _GC)])

            gwait(j, b)
            ostart(j, b)
        return carry

    jax.lax.fori_loop(0, n // 2, outer, 0)
    owait(0)
    owait(1)


def _sc_gather_d2(y, idx, px, py, pz):
    k = pl.kernel(
        _gd2_body,
        out_type=(
            jax.ShapeDtypeStruct((_EDGES, NAB), jnp.float32),
            jax.ShapeDtypeStruct((_EDGES,), jnp.float32),
        ),
        mesh=plsc.VectorSubcoreMesh(core_axis_name="c", subcore_axis_name="s"),
        compiler_params=pltpu.CompilerParams(needs_layout_passes=False),
        scratch_types=[
            pltpu.VMEM((_PER_W,), jnp.int32),
            pltpu.VMEM((2, _GC, NAB), jnp.float32),
            pltpu.VMEM((BA,), jnp.float32),
            pltpu.VMEM((BA,), jnp.float32),
            pltpu.VMEM((BA,), jnp.float32),
            pltpu.VMEM((_GC,), jnp.float32),
            pltpu.SemaphoreType.DMA,
            pltpu.SemaphoreType.DMA,
            pltpu.SemaphoreType.DMA,
            pltpu.SemaphoreType.DMA,
        ],
    )
    return k(y, idx, px, py, pz)


def _ssp(t):
    return jnp.maximum(t, 0.0) + jnp.log1p(jnp.exp(-jnp.abs(t))) - LN2


def _embed_body(z_ref, emb_ref, win_ref, x_ref, y_ref):
    z = z_ref[...]
    col = jax.lax.broadcasted_iota(jnp.int32, (BA, NAB), 1)
    onehot = (z == col).astype(jnp.float32)
    x = jnp.dot(onehot, emb_ref[...], preferred_element_type=jnp.float32)
    x_ref[...] = x
    y_ref[...] = jnp.dot(x, win_ref[...], preferred_element_type=jnp.float32)


def _embed_call(z_flat, emb_pad, win0):
    return pl.pallas_call(
        _embed_body,
        out_shape=(
            jax.ShapeDtypeStruct((BA, NAB), jnp.float32),
            jax.ShapeDtypeStruct((BA, NAB), jnp.float32),
        ),
    )(z_flat, emb_pad, win0)


def _interact_body(d2_ref, yj_ref, x_ref, wfn1_ref, wfn2_ref, wf2out_ref,
                   wdense_ref, winext_ref, xo_ref, yo_ref):
    width = CUTOFF / (NG - 1)
    coeff = -0.5 / (width * width)
    r = jnp.sqrt(jnp.maximum(d2_ref[...], 1e-10))
    offs = jax.lax.broadcasted_iota(
        jnp.int32, (TA, NN, NG), 2).astype(jnp.float32) * width
    diff = r[:, :, None] - offs
    f = jnp.exp(coeff * (diff * diff))
    f2 = f.reshape(TA * NN, NG)
    t1 = _ssp(jnp.dot(f2, wfn1_ref[...], preferred_element_type=jnp.float32))
    wf = jnp.dot(t1, wfn2_ref[...], preferred_element_type=jnp.float32)
    prod = wf * yj_ref[...]
    agg = prod.reshape(TA, NN, NAB).sum(axis=1)
    h = _ssp(jnp.dot(agg, wf2out_ref[...], preferred_element_type=jnp.float32))
    v = jnp.dot(h, wdense_ref[...], preferred_element_type=jnp.float32)
    xo = x_ref[...] + v
    xo_ref[...] = xo
    yo_ref[...] = jnp.dot(xo, winext_ref[...], preferred_element_type=jnp.float32)


def _interact_call(d2, yj, x, wfn1, wfn2, wf2out, wdense, winext):
    full = lambda k: pl.BlockSpec((k, NAB), lambda t: (0, 0))
    return pl.pallas_call(
        _interact_body,
        grid=(BA // TA,),
        in_specs=[
            pl.BlockSpec((TA, NN), lambda t: (t, 0)),
            pl.BlockSpec((TA * NN, NAB), lambda t: (t, 0)),
            pl.BlockSpec((TA, NAB), lambda t: (t, 0)),
            full(NG), full(NF), full(NF), full(NAB), full(NAB),
        ],
        out_specs=(
            pl.BlockSpec((TA, NAB), lambda t: (t, 0)),
            pl.BlockSpec((TA, NAB), lambda t: (t, 0)),
        ),
        out_shape=(
            jax.ShapeDtypeStruct((BA, NAB), jnp.float32),
            jax.ShapeDtypeStruct((BA, NAB), jnp.float32),
        ),
    )(d2, yj, x, wfn1, wfn2, wf2out, wdense, winext)


def kernel(atomic_numbers, positions, cell, cell_offset, neighbors, neighbor_mask,
           embedding, Wfn1, bfn1, Wfn2, bfn2, Win2f, Wf2out, bf2out, Wdense, bdense):
    z_flat = atomic_numbers.reshape(BA, 1).astype(jnp.int32)
    emb_pad = jnp.pad(embedding, ((0, NAB - MAXZ), (0, 0)))
    x, y = _embed_call(z_flat, emb_pad, Win2f[0])

    nbr_flat = (neighbors.astype(jnp.int32)
                + (jnp.arange(B, dtype=jnp.int32) * A)[:, None, None]).reshape(BA * NN)
    pos_flat = positions.reshape(BA, 3)

    for i in range(N_INT):
        if i == 0:
            yj, d2 = _sc_gather_d2(y, nbr_flat, pos_flat[:, 0],
                                   pos_flat[:, 1], pos_flat[:, 2])
            d2 = d2.reshape(BA, NN)
        else:
            yj = _sc_gather(y, nbr_flat)
        winext = Win2f[i + 1] if i + 1 < N_INT else Win2f[i]
        x, y = _interact_call(d2, yj, x, Wfn1[i], Wfn2[i],
                              Wf2out[i], Wdense[i], winext)
    return x.reshape(B, A, NAB)
